# Initial kernel scaffold; baseline (speedup 1.0000x reference)
import jax
import jax.numpy as jnp
from jax.experimental import pallas as pl


def kernel(x, edge_index, W_in, b_in, W_gat, a_gat, g1, b1, Wqkv, bqkv, Wo, bo, g2, b2, Wc, bc):
    raise NotImplementedError("write your pallas kernel here")


# speedup vs baseline: 7.5412x; 7.5412x over previous
import numpy as np
import jax
import jax.numpy as jnp
from jax.experimental import pallas as pl
from jax.experimental.pallas import tpu as pltpu

T = 4
N = 10000
E = 160000
D = 256
H_S = 8
DPH = 32
H_T = 8
NC = 16

NPAD = 10240
NB = 2560
N_NB = NPAD // NB
EB = 640
N_EB = E // EB
RB = 1000
N_RB = (T * N) // RB
NB6 = 1024
N_NB6 = NPAD // NB6

_R = np.zeros((T * H_S, T * D), dtype=np.float32)
for _t in range(T):
    for _h in range(H_S):
        _R[_t * H_S + _h, _t * D + _h * DPH:(_t * D) + (_h + 1) * DPH] = 1.0
_S = np.zeros((D, H_T), dtype=np.float32)
for _h in range(H_T):
    _S[_h * DPH:(_h + 1) * DPH, _h] = 1.0
_S32 = _S.T.copy()
_HMASK = np.zeros((D, H_S), dtype=np.float32)
for _c in range(D):
    _HMASK[_c, _c // DPH] = 1.0
_DIDX = np.arange(D) % DPH
_IDXQ = np.array([h * 96 + j for h in range(H_T) for j in range(32)])
_IDXK = _IDXQ + 32
_IDXV = _IDXQ + 64


def _pos_encoding():
    pos = np.arange(T, dtype=np.float32)[:, None]
    div = np.exp(np.arange(0, D, 2, dtype=np.float32) * (-np.log(10000.0) / D))
    pe = np.zeros((T, D), dtype=np.float32)
    pe[:, 0::2] = np.sin(pos * div)
    pe[:, 1::2] = np.cos(pos * div)
    return pe


_PE = _pos_encoding()


def _ln(z, g, b):
    m = jnp.mean(z, axis=-1, keepdims=True)
    v = jnp.mean((z - m) * (z - m), axis=-1, keepdims=True)
    return (z - m) / jnp.sqrt(v + 1e-5) * g + b


def _elu(z):
    return jnp.where(z > 0, z, jnp.exp(z) - 1.0)


def _k1_body(x_ref, win_ref, bin_ref, wgat_ref, a1_ref, a2_ref,
             h_ref, ht_ref, u_ref, w_ref):
    h = jnp.dot(x_ref[...], win_ref[...], preferred_element_type=jnp.float32)
    h = h + bin_ref[...]
    ht = jnp.dot(h, wgat_ref[...], preferred_element_type=jnp.float32)
    h_ref[...] = h
    ht_ref[...] = ht
    u_ref[...] = jnp.dot(ht, a1_ref[...], preferred_element_type=jnp.float32)
    w_ref[...] = jnp.dot(ht, a2_ref[...], preferred_element_type=jnp.float32)


def _kg_body(src_ref, dst_ref, u_ref, w_ref, ht_ref, r_ref,
             msg_ref, esc_ref, ug_ref, wg_ref):
    nb = pl.program_id(1)

    @pl.when(nb == 0)
    def _():
        ug_ref[...] = jnp.zeros_like(ug_ref)
        wg_ref[...] = jnp.zeros_like(wg_ref)
        msg_ref[...] = jnp.zeros_like(msg_ref)

    ids = jax.lax.broadcasted_iota(jnp.int32, (EB, NB), 1) + nb * NB
    o_src = (src_ref[...] == ids).astype(jnp.float32)
    o_dst = (dst_ref[...] == ids).astype(jnp.float32)
    ug_ref[...] += jnp.dot(o_src, u_ref[...], preferred_element_type=jnp.float32)
    wg_ref[...] += jnp.dot(o_dst, w_ref[...], preferred_element_type=jnp.float32)
    msg_ref[...] += jnp.dot(o_src, ht_ref[...], preferred_element_type=jnp.float32)

    @pl.when(nb == N_NB - 1)
    def _():
        z = ug_ref[...] + wg_ref[...]
        sc = jnp.where(z > 0, z, 0.2 * z)
        e = jnp.exp(sc)
        esc_ref[...] = e
        efull = jnp.dot(e, r_ref[...], preferred_element_type=jnp.float32)
        msg_ref[...] = efull * msg_ref[...]


def _ks_body(dst_ref, msg_ref, esc_ref, r_ref, out_ref, num_ref, den_ref):
    nb = pl.program_id(0)
    eb = pl.program_id(1)

    @pl.when(eb == 0)
    def _():
        num_ref[...] = jnp.zeros_like(num_ref)
        den_ref[...] = jnp.zeros_like(den_ref)

    ids = jax.lax.broadcasted_iota(jnp.int32, (NB, EB), 0) + nb * NB
    o = (ids == dst_ref[...]).astype(jnp.float32)
    num_ref[...] += jnp.dot(o, msg_ref[...], preferred_element_type=jnp.float32)
    den_ref[...] += jnp.dot(o, esc_ref[...], preferred_element_type=jnp.float32)

    @pl.when(eb == N_EB - 1)
    def _():
        inv = 1.0 / (den_ref[...] + 1e-16)
        invfull = jnp.dot(inv, r_ref[...], preferred_element_type=jnp.float32)
        out_ref[...] = num_ref[...] * invfull


def _k6_body(gat_ref, hall_ref, pe_ref, g1_ref, b1_ref,
             wq_ref, bq_ref, wk_ref, bk_ref, wv_ref, bv_ref,
             s_ref, s32_ref, wo_ref, bo_ref, g2_ref, b2_ref,
             wc_ref, bc_ref, out_ref):
    xs = []
    for t in range(T):
        z = gat_ref[:, t * D:(t + 1) * D] + hall_ref[:, t * D:(t + 1) * D]
        z = _elu(_ln(z, g1_ref[...], b1_ref[...]))
        xs.append(z + pe_ref[t:t + 1, :])

    q = jnp.dot(xs[T - 1], wq_ref[...], preferred_element_type=jnp.float32) + bq_ref[...]
    ks_ = []
    vs_ = []
    scs = []
    scale = 1.0 / float(np.sqrt(D // H_T))
    for s in range(T):
        k = jnp.dot(xs[s], wk_ref[...], preferred_element_type=jnp.float32) + bk_ref[...]
        v = jnp.dot(xs[s], wv_ref[...], preferred_element_type=jnp.float32) + bv_ref[...]
        ks_.append(k)
        vs_.append(v)
        scs.append(jnp.dot(q * k, s_ref[...], preferred_element_type=jnp.float32) * scale)

    m = scs[0]
    for s in range(1, T):
        m = jnp.maximum(m, scs[s])
    es = [jnp.exp(sc - m) for sc in scs]
    den = es[0]
    for s in range(1, T):
        den = den + es[s]
    ao = jnp.zeros_like(q)
    for s in range(T):
        wfull = jnp.dot(es[s] / den, s32_ref[...], preferred_element_type=jnp.float32)
        ao = ao + wfull * vs_[s]

    out3 = jnp.dot(ao, wo_ref[...], preferred_element_type=jnp.float32) + bo_ref[...]
    y = _elu(_ln(xs[T - 1] + out3, g2_ref[...], b2_ref[...]))
    out_ref[...] = jnp.dot(y, wc_ref[...], preferred_element_type=jnp.float32) + bc_ref[...]


def kernel(x, edge_index, W_in, b_in, W_gat, a_gat, g1, b1,
           Wqkv, bqkv, Wo, bo, g2, b2, Wc, bc):
    f32 = jnp.float32

    x2 = x.reshape(T * N, D)
    a1c = a_gat[:DPH][_DIDX]
    a2c = a_gat[DPH:][_DIDX]
    A1 = jnp.asarray(_HMASK) * a1c[:, None]
    A2 = jnp.asarray(_HMASK) * a2c[:, None]
    Rm = jnp.asarray(_R)
    Sm = jnp.asarray(_S)
    S32m = jnp.asarray(_S32)
    Wq = Wqkv[:, _IDXQ]
    Wk = Wqkv[:, _IDXK]
    Wv = Wqkv[:, _IDXV]
    bq = bqkv[_IDXQ].reshape(1, D)
    bk = bqkv[_IDXK].reshape(1, D)
    bv = bqkv[_IDXV].reshape(1, D)
    pe = jnp.asarray(_PE)
    Wc_pad = jnp.zeros((D, 128), f32).at[:, :NC].set(Wc)
    bc_pad = jnp.zeros((1, 128), f32).at[0, :NC].set(bc)
    srcT = edge_index[0].reshape(E, 1)
    dstT = edge_index[1].reshape(E, 1)
    dstL = edge_index[1].reshape(1, E)

    full = lambda shp: pl.BlockSpec(shp, lambda *_: tuple(0 for _ in shp))

    h2, ht2, u2, w2 = pl.pallas_call(
        _k1_body,
        grid=(N_RB,),
        in_specs=[
            pl.BlockSpec((RB, D), lambda i: (i, 0)),
            full((D, D)), full((1, D)), full((D, D)),
            full((D, H_S)), full((D, H_S)),
        ],
        out_specs=[
            pl.BlockSpec((RB, D), lambda i: (i, 0)),
            pl.BlockSpec((RB, D), lambda i: (i, 0)),
            pl.BlockSpec((RB, H_S), lambda i: (i, 0)),
            pl.BlockSpec((RB, H_S), lambda i: (i, 0)),
        ],
        out_shape=[
            jax.ShapeDtypeStruct((T * N, D), f32),
            jax.ShapeDtypeStruct((T * N, D), f32),
            jax.ShapeDtypeStruct((T * N, H_S), f32),
            jax.ShapeDtypeStruct((T * N, H_S), f32),
        ],
    )(x2, W_in, b_in.reshape(1, D), W_gat, A1, A2)

    def node_major(a, c):
        a = a.reshape(T, N, c).transpose(1, 0, 2).reshape(N, T * c)
        return jnp.zeros((NPAD, T * c), f32).at[:N].set(a)

    ht_all = node_major(ht2, D)
    u_all = node_major(u2, H_S)
    w_all = node_major(w2, H_S)
    h_all = node_major(h2, D)

    msg, esc = pl.pallas_call(
        _kg_body,
        grid=(N_EB, N_NB),
        in_specs=[
            pl.BlockSpec((EB, 1), lambda e, n: (e, 0)),
            pl.BlockSpec((EB, 1), lambda e, n: (e, 0)),
            pl.BlockSpec((NB, T * H_S), lambda e, n: (n, 0)),
            pl.BlockSpec((NB, T * H_S), lambda e, n: (n, 0)),
            pl.BlockSpec((NB, T * D), lambda e, n: (n, 0)),
            pl.BlockSpec((T * H_S, T * D), lambda e, n: (0, 0)),
        ],
        out_specs=[
            pl.BlockSpec((EB, T * D), lambda e, n: (e, 0)),
            pl.BlockSpec((EB, T * H_S), lambda e, n: (e, 0)),
        ],
        out_shape=[
            jax.ShapeDtypeStruct((E, T * D), f32),
            jax.ShapeDtypeStruct((E, T * H_S), f32),
        ],
        scratch_shapes=[
            pltpu.VMEM((EB, T * H_S), f32),
            pltpu.VMEM((EB, T * H_S), f32),
        ],
    )(srcT, dstT, u_all, w_all, ht_all, Rm)

    gat = pl.pallas_call(
        _ks_body,
        grid=(N_NB, N_EB),
        in_specs=[
            pl.BlockSpec((1, EB), lambda n, e: (0, e)),
            pl.BlockSpec((EB, T * D), lambda n, e: (e, 0)),
            pl.BlockSpec((EB, T * H_S), lambda n, e: (e, 0)),
            pl.BlockSpec((T * H_S, T * D), lambda n, e: (0, 0)),
        ],
        out_specs=pl.BlockSpec((NB, T * D), lambda n, e: (n, 0)),
        out_shape=jax.ShapeDtypeStruct((NPAD, T * D), f32),
        scratch_shapes=[
            pltpu.VMEM((NB, T * D), f32),
            pltpu.VMEM((NB, T * H_S), f32),
        ],
    )(dstL, msg, esc, Rm)

    out = pl.pallas_call(
        _k6_body,
        grid=(N_NB6,),
        in_specs=[
            pl.BlockSpec((NB6, T * D), lambda i: (i, 0)),
            pl.BlockSpec((NB6, T * D), lambda i: (i, 0)),
            full((T, D)), full((1, D)), full((1, D)),
            full((D, D)), full((1, D)), full((D, D)), full((1, D)),
            full((D, D)), full((1, D)),
            full((D, H_T)), full((H_T, D)),
            full((D, D)), full((1, D)), full((1, D)), full((1, D)),
            full((D, 128)), full((1, 128)),
        ],
        out_specs=pl.BlockSpec((NB6, 128), lambda i: (i, 0)),
        out_shape=jax.ShapeDtypeStruct((NPAD, 128), f32),
    )(gat, h_all, pe, g1.reshape(1, D), b1.reshape(1, D),
      Wq, bq, Wk, bk, Wv, bv, Sm, S32m, Wo, bo.reshape(1, D),
      g2.reshape(1, D), b2.reshape(1, D), Wc_pad, bc_pad)

    return out[:N, :NC]


# speedup vs baseline: 7.6417x; 1.0133x over previous
import numpy as np
import jax
import jax.numpy as jnp
from jax.experimental import pallas as pl
from jax.experimental.pallas import tpu as pltpu

T = 4
N = 10000
E = 160000
D = 256
H_S = 8
DPH = 32
H_T = 8
NC = 16

NPAD = 10240
NB = 2560
N_NB = NPAD // NB
EB = 640
N_EB = E // EB
RB = 1000
N_RB = (T * N) // RB
NB6 = 1024
N_NB6 = NPAD // NB6

_R = np.zeros((T * H_S, T * D), dtype=np.float32)
for _t in range(T):
    for _h in range(H_S):
        _R[_t * H_S + _h, _t * D + _h * DPH:(_t * D) + (_h + 1) * DPH] = 1.0
_S = np.zeros((D, H_T), dtype=np.float32)
for _h in range(H_T):
    _S[_h * DPH:(_h + 1) * DPH, _h] = 1.0
_S32 = _S.T.copy()
_HMASK = np.zeros((D, H_S), dtype=np.float32)
for _c in range(D):
    _HMASK[_c, _c // DPH] = 1.0
_DIDX = np.arange(D) % DPH
_IDXQ = np.array([h * 96 + j for h in range(H_T) for j in range(32)])
_IDXK = _IDXQ + 32
_IDXV = _IDXQ + 64


def _pos_encoding():
    pos = np.arange(T, dtype=np.float32)[:, None]
    div = np.exp(np.arange(0, D, 2, dtype=np.float32) * (-np.log(10000.0) / D))
    pe = np.zeros((T, D), dtype=np.float32)
    pe[:, 0::2] = np.sin(pos * div)
    pe[:, 1::2] = np.cos(pos * div)
    return pe


_PE = _pos_encoding()


def _ln(z, g, b):
    m = jnp.mean(z, axis=-1, keepdims=True)
    v = jnp.mean((z - m) * (z - m), axis=-1, keepdims=True)
    return (z - m) / jnp.sqrt(v + 1e-5) * g + b


def _elu(z):
    return jnp.where(z > 0, z, jnp.exp(z) - 1.0)


def _k1_body(x_ref, win_ref, bin_ref, wgat_ref, a1_ref, a2_ref,
             h_ref, ht_ref, u_ref, w_ref):
    h = jnp.dot(x_ref[...], win_ref[...], preferred_element_type=jnp.float32)
    h = h + bin_ref[...]
    ht = jnp.dot(h, wgat_ref[...], preferred_element_type=jnp.float32)
    h_ref[...] = h
    ht_ref[...] = ht
    u_ref[...] = jnp.dot(ht, a1_ref[...], preferred_element_type=jnp.float32)
    w_ref[...] = jnp.dot(ht, a2_ref[...], preferred_element_type=jnp.float32)


def _kg_body(src_ref, dst_ref, u_ref, w_ref, ht_ref, r_ref,
             msg_ref, esc_ref, ug_ref, wg_ref):
    nb = pl.program_id(1)

    @pl.when(nb == 0)
    def _():
        ug_ref[...] = jnp.zeros_like(ug_ref)
        wg_ref[...] = jnp.zeros_like(wg_ref)
        msg_ref[...] = jnp.zeros_like(msg_ref)

    ids = jax.lax.broadcasted_iota(jnp.int32, (EB, NB), 1) + nb * NB
    hit_src = src_ref[...] == ids
    hit_dst = dst_ref[...] == ids
    o_src = hit_src.astype(jnp.float32)
    o_dst = hit_dst.astype(jnp.float32)
    o_srcb = hit_src.astype(jnp.bfloat16)
    ug_ref[...] += jnp.dot(o_src, u_ref[...], preferred_element_type=jnp.float32)
    wg_ref[...] += jnp.dot(o_dst, w_ref[...], preferred_element_type=jnp.float32)
    msg_ref[...] += jnp.dot(o_srcb, ht_ref[...],
                            preferred_element_type=jnp.float32).astype(jnp.bfloat16)

    @pl.when(nb == N_NB - 1)
    def _():
        z = ug_ref[...] + wg_ref[...]
        sc = jnp.where(z > 0, z, 0.2 * z)
        e = jnp.exp(sc)
        esc_ref[...] = e.astype(jnp.bfloat16)
        efull = jnp.dot(e, r_ref[...], preferred_element_type=jnp.float32)
        msg_ref[...] = (efull * msg_ref[...].astype(jnp.float32)).astype(jnp.bfloat16)


def _ks_body(dst_ref, msg_ref, esc_ref, r_ref, out_ref, num_ref, den_ref):
    nb = pl.program_id(0)
    eb = pl.program_id(1)

    @pl.when(eb == 0)
    def _():
        num_ref[...] = jnp.zeros_like(num_ref)
        den_ref[...] = jnp.zeros_like(den_ref)

    ids = jax.lax.broadcasted_iota(jnp.int32, (NB, EB), 0) + nb * NB
    o = (ids == dst_ref[...]).astype(jnp.bfloat16)
    num_ref[...] += jnp.dot(o, msg_ref[...], preferred_element_type=jnp.float32)
    den_ref[...] += jnp.dot(o, esc_ref[...], preferred_element_type=jnp.float32)

    @pl.when(eb == N_EB - 1)
    def _():
        inv = 1.0 / (den_ref[...] + 1e-16)
        invfull = jnp.dot(inv, r_ref[...], preferred_element_type=jnp.float32)
        out_ref[...] = num_ref[...] * invfull


def _k6_body(gat_ref, hall_ref, pe_ref, g1_ref, b1_ref,
             wq_ref, bq_ref, wk_ref, bk_ref, wv_ref, bv_ref,
             s_ref, s32_ref, wo_ref, bo_ref, g2_ref, b2_ref,
             wc_ref, bc_ref, out_ref):
    xs = []
    for t in range(T):
        z = gat_ref[:, t * D:(t + 1) * D] + hall_ref[:, t * D:(t + 1) * D]
        z = _elu(_ln(z, g1_ref[...], b1_ref[...]))
        xs.append(z + pe_ref[t:t + 1, :])

    q = jnp.dot(xs[T - 1], wq_ref[...], preferred_element_type=jnp.float32) + bq_ref[...]
    ks_ = []
    vs_ = []
    scs = []
    scale = 1.0 / float(np.sqrt(D // H_T))
    for s in range(T):
        k = jnp.dot(xs[s], wk_ref[...], preferred_element_type=jnp.float32) + bk_ref[...]
        v = jnp.dot(xs[s], wv_ref[...], preferred_element_type=jnp.float32) + bv_ref[...]
        ks_.append(k)
        vs_.append(v)
        scs.append(jnp.dot(q * k, s_ref[...], preferred_element_type=jnp.float32) * scale)

    m = scs[0]
    for s in range(1, T):
        m = jnp.maximum(m, scs[s])
    es = [jnp.exp(sc - m) for sc in scs]
    den = es[0]
    for s in range(1, T):
        den = den + es[s]
    ao = jnp.zeros_like(q)
    for s in range(T):
        wfull = jnp.dot(es[s] / den, s32_ref[...], preferred_element_type=jnp.float32)
        ao = ao + wfull * vs_[s]

    out3 = jnp.dot(ao, wo_ref[...], preferred_element_type=jnp.float32) + bo_ref[...]
    y = _elu(_ln(xs[T - 1] + out3, g2_ref[...], b2_ref[...]))
    out_ref[...] = jnp.dot(y, wc_ref[...], preferred_element_type=jnp.float32) + bc_ref[...]


def kernel(x, edge_index, W_in, b_in, W_gat, a_gat, g1, b1,
           Wqkv, bqkv, Wo, bo, g2, b2, Wc, bc):
    f32 = jnp.float32

    x2 = x.reshape(T * N, D)
    a1c = a_gat[:DPH][_DIDX]
    a2c = a_gat[DPH:][_DIDX]
    A1 = jnp.asarray(_HMASK) * a1c[:, None]
    A2 = jnp.asarray(_HMASK) * a2c[:, None]
    Rm = jnp.asarray(_R)
    Sm = jnp.asarray(_S)
    S32m = jnp.asarray(_S32)
    Wq = Wqkv[:, _IDXQ]
    Wk = Wqkv[:, _IDXK]
    Wv = Wqkv[:, _IDXV]
    bq = bqkv[_IDXQ].reshape(1, D)
    bk = bqkv[_IDXK].reshape(1, D)
    bv = bqkv[_IDXV].reshape(1, D)
    pe = jnp.asarray(_PE)
    Wc_pad = jnp.zeros((D, 128), f32).at[:, :NC].set(Wc)
    bc_pad = jnp.zeros((1, 128), f32).at[0, :NC].set(bc)
    srcT = edge_index[0].reshape(E, 1)
    dstT = edge_index[1].reshape(E, 1)
    dstL = edge_index[1].reshape(1, E)

    full = lambda shp: pl.BlockSpec(shp, lambda *_: tuple(0 for _ in shp))

    h2, ht2, u2, w2 = pl.pallas_call(
        _k1_body,
        grid=(N_RB,),
        in_specs=[
            pl.BlockSpec((RB, D), lambda i: (i, 0)),
            full((D, D)), full((1, D)), full((D, D)),
            full((D, H_S)), full((D, H_S)),
        ],
        out_specs=[
            pl.BlockSpec((RB, D), lambda i: (i, 0)),
            pl.BlockSpec((RB, D), lambda i: (i, 0)),
            pl.BlockSpec((RB, H_S), lambda i: (i, 0)),
            pl.BlockSpec((RB, H_S), lambda i: (i, 0)),
        ],
        out_shape=[
            jax.ShapeDtypeStruct((T * N, D), f32),
            jax.ShapeDtypeStruct((T * N, D), f32),
            jax.ShapeDtypeStruct((T * N, H_S), f32),
            jax.ShapeDtypeStruct((T * N, H_S), f32),
        ],
    )(x2, W_in, b_in.reshape(1, D), W_gat, A1, A2)

    def node_major(a, c):
        a = a.reshape(T, N, c).transpose(1, 0, 2).reshape(N, T * c)
        return jnp.zeros((NPAD, T * c), f32).at[:N].set(a)

    ht_all = node_major(ht2, D)
    u_all = node_major(u2, H_S)
    w_all = node_major(w2, H_S)
    h_all = node_major(h2, D)

    msg, esc = pl.pallas_call(
        _kg_body,
        grid=(N_EB, N_NB),
        in_specs=[
            pl.BlockSpec((EB, 1), lambda e, n: (e, 0)),
            pl.BlockSpec((EB, 1), lambda e, n: (e, 0)),
            pl.BlockSpec((NB, T * H_S), lambda e, n: (n, 0)),
            pl.BlockSpec((NB, T * H_S), lambda e, n: (n, 0)),
            pl.BlockSpec((NB, T * D), lambda e, n: (n, 0)),
            pl.BlockSpec((T * H_S, T * D), lambda e, n: (0, 0)),
        ],
        out_specs=[
            pl.BlockSpec((EB, T * D), lambda e, n: (e, 0)),
            pl.BlockSpec((EB, T * H_S), lambda e, n: (e, 0)),
        ],
        out_shape=[
            jax.ShapeDtypeStruct((E, T * D), jnp.bfloat16),
            jax.ShapeDtypeStruct((E, T * H_S), jnp.bfloat16),
        ],
        scratch_shapes=[
            pltpu.VMEM((EB, T * H_S), f32),
            pltpu.VMEM((EB, T * H_S), f32),
        ],
    )(srcT, dstT, u_all, w_all, ht_all.astype(jnp.bfloat16), Rm)

    gat = pl.pallas_call(
        _ks_body,
        grid=(N_NB, N_EB),
        in_specs=[
            pl.BlockSpec((1, EB), lambda n, e: (0, e)),
            pl.BlockSpec((EB, T * D), lambda n, e: (e, 0)),
            pl.BlockSpec((EB, T * H_S), lambda n, e: (e, 0)),
            pl.BlockSpec((T * H_S, T * D), lambda n, e: (0, 0)),
        ],
        out_specs=pl.BlockSpec((NB, T * D), lambda n, e: (n, 0)),
        out_shape=jax.ShapeDtypeStruct((NPAD, T * D), f32),
        scratch_shapes=[
            pltpu.VMEM((NB, T * D), f32),
            pltpu.VMEM((NB, T * H_S), f32),
        ],
    )(dstL, msg, esc, Rm)

    out = pl.pallas_call(
        _k6_body,
        grid=(N_NB6,),
        in_specs=[
            pl.BlockSpec((NB6, T * D), lambda i: (i, 0)),
            pl.BlockSpec((NB6, T * D), lambda i: (i, 0)),
            full((T, D)), full((1, D)), full((1, D)),
            full((D, D)), full((1, D)), full((D, D)), full((1, D)),
            full((D, D)), full((1, D)),
            full((D, H_T)), full((H_T, D)),
            full((D, D)), full((1, D)), full((1, D)), full((1, D)),
            full((D, 128)), full((1, 128)),
        ],
        out_specs=pl.BlockSpec((NB6, 128), lambda i: (i, 0)),
        out_shape=jax.ShapeDtypeStruct((NPAD, 128), f32),
    )(gat, h_all, pe, g1.reshape(1, D), b1.reshape(1, D),
      Wq, bq, Wk, bk, Wv, bv, Sm, S32m, Wo, bo.reshape(1, D),
      g2.reshape(1, D), b2.reshape(1, D), Wc_pad, bc_pad)

    return out[:N, :NC]
